# baseline (speedup 1.0000x reference)
import functools

import jax
import jax.numpy as jnp
from jax import lax
from jax.experimental import pallas as pl
from jax.experimental.pallas import tpu as pltpu
from jax.experimental.pallas import tpu_sc as plsc

NC = 2
NS = 16
NW = NC * NS
CH = 80
SPLITS = (38400, 102400, 89600, 51200, 38400)


def _sc_mesh():
    return plsc.VectorSubcoreMesh(core_axis_name="c", subcore_axis_name="s")


def _gather_pairs(atom_fea, src, dst):
    E = src.shape[0]
    N, D = atom_fea.shape
    per_w = E // NW
    n_ch = per_w // CH

    @functools.partial(
        pl.kernel,
        out_type=(
            jax.ShapeDtypeStruct((E, D), jnp.float32),
            jax.ShapeDtypeStruct((E, D), jnp.float32),
        ),
        mesh=_sc_mesh(),
        scratch_types=[
            pltpu.VMEM((per_w,), jnp.int32),
            pltpu.VMEM((per_w,), jnp.int32),
            pltpu.VMEM((2, CH, D), jnp.float32),
            pltpu.VMEM((2, CH, D), jnp.float32),
            pltpu.VMEM_SHARED((N, D), jnp.float32),
            pltpu.SemaphoreType.DMA,
            pltpu.SemaphoreType.DMA,
        ],
    )
    def k(atom_hbm, src_hbm, dst_hbm, gs_hbm, gd_hbm,
          idx_s, idx_d, rows_s, rows_d, atom_sp, sem_g, sem_w):
        sid = lax.axis_index("s")
        wid = sid * NC + lax.axis_index("c")
        base = wid * per_w
        pltpu.sync_copy(src_hbm.at[pl.ds(base, per_w)], idx_s)
        pltpu.sync_copy(dst_hbm.at[pl.ds(base, per_w)], idx_d)

        @pl.when(sid == 0)
        def _():
            pltpu.sync_copy(atom_hbm, atom_sp)

        plsc.subcore_barrier()

        def fire_gathers(j, p):
            off = j * CH
            pltpu.async_copy(atom_sp.at[idx_s.at[pl.ds(off, CH)]],
                             rows_s.at[p], sem_g)
            pltpu.async_copy(atom_sp.at[idx_d.at[pl.ds(off, CH)]],
                             rows_d.at[p], sem_g)

        fire_gathers(0, 0)

        def body(j, carry):
            p = lax.rem(j, 2)
            pltpu.make_async_copy(gs_hbm.at[pl.ds(0, CH)], rows_s.at[p], sem_g).wait()
            pltpu.make_async_copy(gs_hbm.at[pl.ds(0, CH)], rows_d.at[p], sem_g).wait()
            pltpu.async_copy(rows_s.at[p], gs_hbm.at[pl.ds(base + j * CH, CH)], sem_w)
            pltpu.async_copy(rows_d.at[p], gd_hbm.at[pl.ds(base + j * CH, CH)], sem_w)

            @pl.when(j >= 1)
            def _():
                pltpu.make_async_copy(gs_hbm.at[pl.ds(0, CH)],
                                      rows_s.at[1 - p], sem_w).wait()
                pltpu.make_async_copy(gs_hbm.at[pl.ds(0, CH)],
                                      rows_d.at[1 - p], sem_w).wait()

            @pl.when(j < n_ch - 1)
            def _():
                fire_gathers(j + 1, 1 - p)

            return carry

        lax.fori_loop(0, n_ch, body, 0)
        pltpu.make_async_copy(gs_hbm.at[pl.ds(0, CH)],
                              rows_s.at[lax.rem(n_ch - 1, 2)], sem_w).wait()
        pltpu.make_async_copy(gs_hbm.at[pl.ds(0, CH)],
                              rows_d.at[lax.rem(n_ch - 1, 2)], sem_w).wait()

    return k(atom_fea, src, dst)


def _edge_mlp(gs, gd, eij, bor, e_off, Ws, Wd, We, bf, W1, b1, W2, b2,
              Wr, br):
    E, D = gs.shape
    NFE = eij.shape[0]
    NFB = bor.shape[0]
    BE = 12800
    grid = (E // BE,)
    blk0 = e_off // BE

    def sig(x):
        return 0.5 * jnp.tanh(0.5 * x) + 0.5

    def body(gs_r, gd_r, eij_r, bor_r, Ws_r, Wd_r, We_r, bf_r,
             W1_r, b1_r, W2_r, b2_r, Wr_r, br_r, out_r):
        dot = functools.partial(jnp.dot, preferred_element_type=jnp.float32)
        dot_t = functools.partial(
            lax.dot_general, dimension_numbers=(((0,), (0,)), ((), ())),
            preferred_element_type=jnp.float32)
        h = (dot(gs_r[...].astype(jnp.bfloat16), Ws_r[...])
             + dot(gd_r[...].astype(jnp.bfloat16), Wd_r[...])
             + dot_t(eij_r[...].astype(jnp.bfloat16), We_r[...]) + bf_r[...])
        h = (h * sig(h)).astype(jnp.bfloat16)
        g1 = dot(h, W1_r[...]) + b1_r[...]
        g2 = dot(h, W2_r[...]) + b2_r[...]
        r = dot_t(bor_r[...].astype(jnp.bfloat16), Wr_r[...]) + br_r[...]
        out_r[...] = g1 * sig(g1) * sig(g2) * r

    eb = lambda w: pl.BlockSpec((BE, w), lambda i: (i, 0))
    ofs = lambda w: pl.BlockSpec((w, BE), lambda i: (0, blk0 + i))
    full = lambda a: pl.BlockSpec(a.shape, lambda i: (0,) * a.ndim)
    return pl.pallas_call(
        body,
        grid=grid,
        in_specs=[eb(D), eb(D), ofs(NFE), ofs(NFB),
                  full(Ws), full(Wd), full(We), full(bf),
                  full(W1), full(b1), full(W2), full(b2), full(Wr), full(br)],
        out_specs=eb(D),
        out_shape=jax.ShapeDtypeStruct((E, D), jnp.float32),
    )(gs, gd, eij, bor, Ws, Wd, We, bf, W1, b1, W2, b2, Wr, br)


def _scatter_add_split(nbr, src3, prev, n_nodes):
    E_c, D = nbr.shape
    per_w = E_c // NW
    n_ch = per_w // CH

    @functools.partial(
        pl.kernel,
        out_type=jax.ShapeDtypeStruct((NC, n_nodes, D), jnp.float32),
        mesh=_sc_mesh(),
        scratch_types=[
            pltpu.VMEM((n_ch, CH), jnp.int32),
            pltpu.VMEM((2, CH, D), jnp.float32),
            pltpu.VMEM_SHARED((n_nodes, D), jnp.float32),
            pltpu.SemaphoreType.DMA,
            pltpu.SemaphoreType.DMA,
        ],
    )
    def k(nbr_hbm, src3_hbm, prev_hbm, out_hbm, idx2, rows, acc,
          sem_r, sem_s):
        cid = lax.axis_index("c")
        sid = lax.axis_index("s")
        wid = sid * NC + cid

        @pl.when(sid == 0)
        def _():
            pltpu.sync_copy(prev_hbm.at[cid], acc)

        pltpu.sync_copy(src3_hbm.at[wid], idx2)
        plsc.subcore_barrier()

        base = wid * per_w
        pltpu.async_copy(nbr_hbm.at[pl.ds(base, CH)], rows.at[0], sem_r)

        def body(j, carry):
            p = lax.rem(j, 2)
            pltpu.make_async_copy(nbr_hbm.at[pl.ds(0, CH)], rows.at[p],
                                  sem_r).wait()
            pltpu.async_copy(rows.at[p], acc.at[idx2.at[j]], sem_s, add=True)

            @pl.when(j >= 1)
            def _():
                pltpu.make_async_copy(nbr_hbm.at[pl.ds(0, CH)],
                                      rows.at[1 - p], sem_s).wait()

            @pl.when(j < n_ch - 1)
            def _():
                pltpu.async_copy(nbr_hbm.at[pl.ds(base + (j + 1) * CH, CH)],
                                 rows.at[1 - p], sem_r)

            return carry

        lax.fori_loop(0, n_ch, body, 0)
        pltpu.make_async_copy(nbr_hbm.at[pl.ds(0, CH)],
                              rows.at[lax.rem(n_ch - 1, 2)], sem_s).wait()

        plsc.subcore_barrier()

        @pl.when(sid == 0)
        def _():
            pltpu.sync_copy(acc, out_hbm.at[cid])

    return k(nbr, src3, prev)


def _combine(atom_fea, parts):
    N, D = atom_fea.shape
    BN = 1000

    def body(a_r, *refs):
        p_refs, o_r = refs[:-1], refs[-1]
        o_r[...] = a_r[...] + sum(p_r[0] + p_r[1] for p_r in p_refs)

    spec = pl.BlockSpec((BN, D), lambda i: (i, 0))
    pspec = pl.BlockSpec((NC, BN, D), lambda i: (0, i, 0))
    return pl.pallas_call(
        body,
        grid=(N // BN,),
        in_specs=[spec] + [pspec] * len(parts),
        out_specs=spec,
        out_shape=jax.ShapeDtypeStruct((N, D), jnp.float32),
    )(atom_fea, *parts)


def kernel(atom_fea, edge_ij, bonds_r, nbr_atoms, W_full, b_full,
           W1, b1, W2, b2, Wr, br):
    n_nodes, D = atom_fea.shape
    E = nbr_atoms.shape[0]
    src = nbr_atoms[:, 0]
    dst = nbr_atoms[:, 1]

    bf16 = lambda a: a.astype(jnp.bfloat16)
    Ws = W_full[:D]
    Wd = W_full[D:2 * D]
    We = W_full[2 * D:]
    mlp_args = (
        bf16(Ws), bf16(Wd), bf16(We), b_full.reshape(1, -1),
        bf16(W1), b1.reshape(1, -1), bf16(W2), b2.reshape(1, -1),
        bf16(Wr), br.reshape(1, -1),
    )

    eij_t = edge_ij.T
    bor_t = bonds_r.T
    nbrs = []
    src3s = []
    off = 0
    for e_c in SPLITS:
        sl = slice(off, off + e_c)
        gs, gd = _gather_pairs(atom_fea, src[sl], dst[sl])
        nbrs.append(_edge_mlp(gs, gd, eij_t, bor_t, off, *mlp_args))
        src3s.append(src[sl].reshape(NW, (e_c // NW) // CH, CH))
        off += e_c

    zeros = jnp.zeros((NC, n_nodes, D), dtype=jnp.float32)
    p_a = zeros
    for c in (0, 2, 4):
        p_a = _scatter_add_split(nbrs[c], src3s[c], p_a, n_nodes)
    p_b = zeros
    for c in (1, 3):
        p_b = _scatter_add_split(nbrs[c], src3s[c], p_b, n_nodes)
    return _combine(atom_fea, [p_a, p_b])


# speedup vs baseline: 1.0535x; 1.0535x over previous
import functools

import jax
import jax.numpy as jnp
from jax import lax
from jax.experimental import pallas as pl
from jax.experimental.pallas import tpu as pltpu
from jax.experimental.pallas import tpu_sc as plsc

NC = 2
NS = 16
NW = NC * NS
CH = 80
SPLITS = (38400, 102400, 89600, 51200, 38400)


def _sc_mesh():
    return plsc.VectorSubcoreMesh(core_axis_name="c", subcore_axis_name="s")


def _gather_pairs(atom_fea, src, dst):
    E = src.shape[0]
    N, D = atom_fea.shape
    per_w = E // NW
    n_ch = per_w // CH

    @functools.partial(
        pl.kernel,
        out_type=(
            jax.ShapeDtypeStruct((E, D), jnp.float32),
            jax.ShapeDtypeStruct((E, D), jnp.float32),
        ),
        mesh=_sc_mesh(),
        scratch_types=[
            pltpu.VMEM((per_w,), jnp.int32),
            pltpu.VMEM((per_w,), jnp.int32),
            pltpu.VMEM((2, CH, D), jnp.float32),
            pltpu.VMEM((2, CH, D), jnp.float32),
            pltpu.VMEM_SHARED((N, D), jnp.float32),
            pltpu.SemaphoreType.DMA,
            pltpu.SemaphoreType.DMA,
        ],
    )
    def k(atom_hbm, src_hbm, dst_hbm, gs_hbm, gd_hbm,
          idx_s, idx_d, rows_s, rows_d, atom_sp, sem_g, sem_w):
        sid = lax.axis_index("s")
        wid = sid * NC + lax.axis_index("c")
        base = wid * per_w
        pltpu.sync_copy(src_hbm.at[pl.ds(base, per_w)], idx_s)
        pltpu.sync_copy(dst_hbm.at[pl.ds(base, per_w)], idx_d)

        @pl.when(sid == 0)
        def _():
            pltpu.sync_copy(atom_hbm, atom_sp)

        plsc.subcore_barrier()

        def fire_gathers(j, p):
            off = j * CH
            pltpu.async_copy(atom_sp.at[idx_s.at[pl.ds(off, CH)]],
                             rows_s.at[p], sem_g)
            pltpu.async_copy(atom_sp.at[idx_d.at[pl.ds(off, CH)]],
                             rows_d.at[p], sem_g)

        fire_gathers(0, 0)

        def body(j, carry):
            p = lax.rem(j, 2)
            pltpu.make_async_copy(gs_hbm.at[pl.ds(0, CH)], rows_s.at[p], sem_g).wait()
            pltpu.make_async_copy(gs_hbm.at[pl.ds(0, CH)], rows_d.at[p], sem_g).wait()
            pltpu.async_copy(rows_s.at[p], gs_hbm.at[pl.ds(base + j * CH, CH)], sem_w)
            pltpu.async_copy(rows_d.at[p], gd_hbm.at[pl.ds(base + j * CH, CH)], sem_w)

            @pl.when(j >= 1)
            def _():
                pltpu.make_async_copy(gs_hbm.at[pl.ds(0, CH)],
                                      rows_s.at[1 - p], sem_w).wait()
                pltpu.make_async_copy(gs_hbm.at[pl.ds(0, CH)],
                                      rows_d.at[1 - p], sem_w).wait()

            @pl.when(j < n_ch - 1)
            def _():
                fire_gathers(j + 1, 1 - p)

            return carry

        lax.fori_loop(0, n_ch, body, 0)
        pltpu.make_async_copy(gs_hbm.at[pl.ds(0, CH)],
                              rows_s.at[lax.rem(n_ch - 1, 2)], sem_w).wait()
        pltpu.make_async_copy(gs_hbm.at[pl.ds(0, CH)],
                              rows_d.at[lax.rem(n_ch - 1, 2)], sem_w).wait()

    return k(atom_fea, src, dst)


def _edge_mlp(gs, gd, eij, bor, e_off, Ws, Wd, We, bf, W1, b1, W2, b2,
              Wr, br):
    E, D = gs.shape
    NFE = eij.shape[0]
    NFB = bor.shape[0]
    BE = 6400
    grid = (E // BE,)
    blk0 = e_off // BE

    def sig(x):
        return 0.5 * jnp.tanh(0.5 * x) + 0.5

    def body(gs_r, gd_r, eij_r, bor_r, Ws_r, Wd_r, We_r, bf_r,
             W1_r, b1_r, W2_r, b2_r, Wr_r, br_r, out_r):
        dot = functools.partial(jnp.dot, preferred_element_type=jnp.float32)
        dot_t = functools.partial(
            lax.dot_general, dimension_numbers=(((0,), (0,)), ((), ())),
            preferred_element_type=jnp.float32)
        h = (dot(gs_r[...].astype(jnp.bfloat16), Ws_r[...])
             + dot(gd_r[...].astype(jnp.bfloat16), Wd_r[...])
             + dot_t(eij_r[...].astype(jnp.bfloat16), We_r[...]) + bf_r[...])
        h = (h * sig(h)).astype(jnp.bfloat16)
        g1 = dot(h, W1_r[...]) + b1_r[...]
        g2 = dot(h, W2_r[...]) + b2_r[...]
        r = dot_t(bor_r[...].astype(jnp.bfloat16), Wr_r[...]) + br_r[...]
        out_r[...] = g1 * sig(g1) * sig(g2) * r

    eb = lambda w: pl.BlockSpec((BE, w), lambda i: (i, 0))
    ofs = lambda w: pl.BlockSpec((w, BE), lambda i: (0, blk0 + i))
    full = lambda a: pl.BlockSpec(a.shape, lambda i: (0,) * a.ndim)
    return pl.pallas_call(
        body,
        grid=grid,
        in_specs=[eb(D), eb(D), ofs(NFE), ofs(NFB),
                  full(Ws), full(Wd), full(We), full(bf),
                  full(W1), full(b1), full(W2), full(b2), full(Wr), full(br)],
        out_specs=eb(D),
        out_shape=jax.ShapeDtypeStruct((E, D), jnp.float32),
    )(gs, gd, eij, bor, Ws, Wd, We, bf, W1, b1, W2, b2, Wr, br)


def _scatter_add_split(nbr, src3, prev, n_nodes):
    E_c, D = nbr.shape
    per_w = E_c // NW
    n_ch = per_w // CH

    @functools.partial(
        pl.kernel,
        out_type=jax.ShapeDtypeStruct((NC, n_nodes, D), jnp.float32),
        mesh=_sc_mesh(),
        scratch_types=[
            pltpu.VMEM((n_ch, CH), jnp.int32),
            pltpu.VMEM((2, CH, D), jnp.float32),
            pltpu.VMEM_SHARED((n_nodes, D), jnp.float32),
            pltpu.SemaphoreType.DMA,
            pltpu.SemaphoreType.DMA,
        ],
    )
    def k(nbr_hbm, src3_hbm, prev_hbm, out_hbm, idx2, rows, acc,
          sem_r, sem_s):
        cid = lax.axis_index("c")
        sid = lax.axis_index("s")
        wid = sid * NC + cid

        @pl.when(sid == 0)
        def _():
            pltpu.sync_copy(prev_hbm.at[cid], acc)

        pltpu.sync_copy(src3_hbm.at[wid], idx2)
        plsc.subcore_barrier()

        base = wid * per_w
        pltpu.async_copy(nbr_hbm.at[pl.ds(base, CH)], rows.at[0], sem_r)

        def body(j, carry):
            p = lax.rem(j, 2)
            pltpu.make_async_copy(nbr_hbm.at[pl.ds(0, CH)], rows.at[p],
                                  sem_r).wait()
            pltpu.async_copy(rows.at[p], acc.at[idx2.at[j]], sem_s, add=True)

            @pl.when(j >= 1)
            def _():
                pltpu.make_async_copy(nbr_hbm.at[pl.ds(0, CH)],
                                      rows.at[1 - p], sem_s).wait()

            @pl.when(j < n_ch - 1)
            def _():
                pltpu.async_copy(nbr_hbm.at[pl.ds(base + (j + 1) * CH, CH)],
                                 rows.at[1 - p], sem_r)

            return carry

        lax.fori_loop(0, n_ch, body, 0)
        pltpu.make_async_copy(nbr_hbm.at[pl.ds(0, CH)],
                              rows.at[lax.rem(n_ch - 1, 2)], sem_s).wait()

        plsc.subcore_barrier()

        @pl.when(sid == 0)
        def _():
            pltpu.sync_copy(acc, out_hbm.at[cid])

    return k(nbr, src3, prev)


def _combine(atom_fea, parts):
    N, D = atom_fea.shape
    BN = 1000

    def body(a_r, *refs):
        p_refs, o_r = refs[:-1], refs[-1]
        o_r[...] = a_r[...] + sum(p_r[0] + p_r[1] for p_r in p_refs)

    spec = pl.BlockSpec((BN, D), lambda i: (i, 0))
    pspec = pl.BlockSpec((NC, BN, D), lambda i: (0, i, 0))
    return pl.pallas_call(
        body,
        grid=(N // BN,),
        in_specs=[spec] + [pspec] * len(parts),
        out_specs=spec,
        out_shape=jax.ShapeDtypeStruct((N, D), jnp.float32),
    )(atom_fea, *parts)


def kernel(atom_fea, edge_ij, bonds_r, nbr_atoms, W_full, b_full,
           W1, b1, W2, b2, Wr, br):
    n_nodes, D = atom_fea.shape
    E = nbr_atoms.shape[0]
    src = nbr_atoms[:, 0]
    dst = nbr_atoms[:, 1]

    bf16 = lambda a: a.astype(jnp.bfloat16)
    Ws = W_full[:D]
    Wd = W_full[D:2 * D]
    We = W_full[2 * D:]
    mlp_args = (
        bf16(Ws), bf16(Wd), bf16(We), b_full.reshape(1, -1),
        bf16(W1), b1.reshape(1, -1), bf16(W2), b2.reshape(1, -1),
        bf16(Wr), br.reshape(1, -1),
    )

    eij_t = edge_ij.T
    bor_t = bonds_r.T
    nbrs = []
    src3s = []
    off = 0
    for e_c in SPLITS:
        sl = slice(off, off + e_c)
        gs, gd = _gather_pairs(atom_fea, src[sl], dst[sl])
        nbrs.append(_edge_mlp(gs, gd, eij_t, bor_t, off, *mlp_args))
        src3s.append(src[sl].reshape(NW, (e_c // NW) // CH, CH))
        off += e_c

    zeros = jnp.zeros((NC, n_nodes, D), dtype=jnp.float32)
    p_a = zeros
    for c in (0, 2, 4):
        p_a = _scatter_add_split(nbrs[c], src3s[c], p_a, n_nodes)
    p_b = zeros
    for c in (1, 3):
        p_b = _scatter_add_split(nbrs[c], src3s[c], p_b, n_nodes)
    return _combine(atom_fea, [p_a, p_b])
